# Initial kernel scaffold; baseline (speedup 1.0000x reference)
import jax
import jax.numpy as jnp
from jax.experimental import pallas as pl


def kernel(x, edge_index, W1, att_src1, att_dst1, b1, W2, att_src2, att_dst2, b2):
    raise NotImplementedError("write your pallas kernel here")


# speedup vs baseline: 11.7795x; 11.7795x over previous
import functools

import jax
import jax.numpy as jnp
from jax import lax
from jax.experimental import pallas as pl
from jax.experimental.pallas import tpu as pltpu
from jax.experimental.pallas import tpu_sc as plsc

F32 = jnp.float32
I32 = jnp.int32

_NC = 2
_NS = 16
_RB = 1024


def _ceil_to(v, m):
    return (v + m - 1) // m * m


def _tc_feats(x_in, w_pad, att_s, att_d, one_col, n_pad):
    k_in = x_in.shape[1]
    d_pad = w_pad.shape[1]
    grid = n_pad // _RB

    def body(x_ref, w_ref, s_ref, d_ref, h_ref, as_ref, ad_ref, bm_ref):
        h = jnp.dot(x_ref[...], w_ref[...], preferred_element_type=F32)
        col = lax.broadcasted_iota(I32, (_RB, d_pad), 1)
        h = h + (col == one_col).astype(F32)
        h_ref[...] = h
        a_s = jnp.dot(h, s_ref[...], preferred_element_type=F32)
        a_d = jnp.dot(h, d_ref[...], preferred_element_type=F32)
        as_ref[...] = a_s
        ad_ref[...] = a_d
        i0 = lax.broadcasted_iota(I32, (1, 1, 128), 2)
        bm_ref[...] = jnp.where(i0 == 0, jnp.max(a_s),
                                jnp.where(i0 == 1, jnp.max(a_d), -1e30))

    return pl.pallas_call(
        body,
        grid=(grid,),
        in_specs=[pl.BlockSpec((_RB, k_in), lambda i: (i, 0)),
                  pl.BlockSpec((k_in, d_pad), lambda i: (0, 0)),
                  pl.BlockSpec((d_pad, 1), lambda i: (0, 0)),
                  pl.BlockSpec((d_pad, 1), lambda i: (0, 0))],
        out_specs=[pl.BlockSpec((_RB, d_pad), lambda i: (i, 0)),
                   pl.BlockSpec((_RB, 1), lambda i: (i, 0)),
                   pl.BlockSpec((_RB, 1), lambda i: (i, 0)),
                   pl.BlockSpec((1, 1, 128), lambda i: (i, 0, 0))],
        out_shape=[jax.ShapeDtypeStruct((n_pad, d_pad), F32),
                   jax.ShapeDtypeStruct((n_pad, 1), F32),
                   jax.ShapeDtypeStruct((n_pad, 1), F32),
                   jax.ShapeDtypeStruct((grid, 1, 128), F32)],
    )(x_in, w_pad, att_s, att_d)


def _tc_norm(accp, bias_pad, n_pad, d_pad, one_col):
    grid = n_pad // _RB

    n_cc = d_pad // 16

    def body(a_ref, b_ref, o_ref):
        accc = a_ref[0] + a_ref[1]
        acc = jnp.concatenate([accc[c] for c in range(n_cc)], axis=-1)
        col = lax.broadcasted_iota(I32, (_RB, d_pad), 1)
        s = jnp.sum(jnp.where(col == one_col, acc, 0.0), axis=1, keepdims=True)
        o_ref[...] = jnp.maximum(acc / (s + 1e-16) + b_ref[...], 0.0)

    return pl.pallas_call(
        body,
        grid=(grid,),
        in_specs=[pl.BlockSpec((2, n_cc, _RB, 16), lambda i: (0, 0, i, 0)),
                  pl.BlockSpec((1, d_pad), lambda i: (0, 0))],
        out_specs=pl.BlockSpec((_RB, d_pad), lambda i: (i, 0)),
        out_shape=jax.ShapeDtypeStruct((n_pad, d_pad), F32),
    )(accp, bias_pad)


def _sc_logits(a_src, a_dst, src, dst, cvec, n_pad, e_pad, e_real):
    t_per = e_pad // (_NC * _NS)
    n_chunks = t_per // 1024
    mesh = plsc.VectorSubcoreMesh(core_axis_name="c", subcore_axis_name="s",
                                  num_cores=_NC, num_subcores=_NS)

    @functools.partial(
        pl.kernel,
        out_type=jax.ShapeDtypeStruct((e_pad,), F32),
        mesh=mesh,
        compiler_params=pltpu.CompilerParams(needs_layout_passes=False),
        scratch_types=[pltpu.VMEM((n_pad,), F32),
                       pltpu.VMEM((n_pad,), F32),
                       pltpu.VMEM((1024,), I32),
                       pltpu.VMEM((1024,), I32),
                       pltpu.VMEM((1024,), F32),
                       pltpu.VMEM((16,), F32)])
    def k(as_hbm, ad_hbm, src_hbm, dst_hbm, cv_hbm, p_hbm,
          as_v, ad_v, src_v, dst_v, p_v, cv_v):
        cid = lax.axis_index("c")
        sid = lax.axis_index("s")
        wid = cid * _NS + sid
        pltpu.sync_copy(as_hbm, as_v)
        pltpu.sync_copy(ad_hbm, ad_v)
        pltpu.sync_copy(cv_hbm, cv_v)
        cvr = cv_v[...]
        tbase = wid * t_per

        def chunk(g, carry):
            base = tbase + g * 1024
            pltpu.sync_copy(src_hbm.at[pl.ds(base, 1024)], src_v)
            pltpu.sync_copy(dst_hbm.at[pl.ds(base, 1024)], dst_v)

            def grp(i, c2):
                sv = src_v[pl.ds(i * 16, 16)]
                dv = dst_v[pl.ds(i * 16, 16)]
                e = plsc.load_gather(as_v, [sv]) + plsc.load_gather(ad_v, [dv])
                e = jnp.where(e > 0, e, 0.2 * e)
                gi = base + i * 16 + lax.iota(I32, 16)
                p_v[pl.ds(i * 16, 16)] = jnp.where(
                    gi < e_real, jnp.exp(e - cvr), 0.0)
                return c2

            lax.fori_loop(0, 64, grp, 0)
            pltpu.sync_copy(p_v, p_hbm.at[pl.ds(base, 1024)])
            return carry

        lax.fori_loop(0, n_chunks, chunk, 0)

    return k(a_src, a_dst, src, dst, cvec)


def _sc_scatter(hflat, srcoff, dst2, p, n_pad, e_pad, n_cc):
    t_per = e_pad // (_NC * _NS)
    n_chunks = t_per // 1024
    rpt = n_pad // _NS
    zr = 112
    nz = rpt // zr
    mesh = plsc.VectorSubcoreMesh(core_axis_name="c", subcore_axis_name="s",
                                  num_cores=_NC, num_subcores=_NS)

    @functools.partial(
        pl.kernel,
        out_type=jax.ShapeDtypeStruct((_NC, n_cc, n_pad, 16), F32),
        mesh=mesh,
        compiler_params=pltpu.CompilerParams(needs_layout_passes=False,
                                             use_tc_tiling_on_sc=False),
        scratch_types=[pltpu.VMEM((8, 128), I32),
                       pltpu.VMEM((8, 128), I32),
                       pltpu.VMEM((1024,), F32),
                       pltpu.VMEM((128, 16), F32),
                       pltpu.VMEM((zr, 16), F32),
                       pltpu.VMEM_SHARED((n_pad, 16), F32),
                       pltpu.SemaphoreType.DMA])
    def k(h_hbm, so_hbm, d2_hbm, p_hbm, out_hbm,
          srcb, dstb, pb, rows, zbuf, acc, sem):
        cid = lax.axis_index("c")
        sid = lax.axis_index("s")
        wid = cid * _NS + sid
        row0 = sid * rpt

        def zrow(r, c_):
            zbuf[r] = jnp.zeros((16,), F32)
            return c_

        lax.fori_loop(0, zr, zrow, 0)

        for c in range(n_cc):
            plsc.subcore_barrier()

            def zcp(z, c_):
                pltpu.sync_copy(zbuf, acc.at[pl.ds(row0 + z * zr, zr)])
                return c_

            lax.fori_loop(0, nz, zcp, 0)
            plsc.subcore_barrier()

            def chunk(g, c_):
                ebase = wid * t_per + g * 1024
                rbase = wid * (t_per // 128) + g * 8
                pltpu.sync_copy(so_hbm.at[c, pl.ds(rbase, 8)], srcb)
                pltpu.sync_copy(d2_hbm.at[pl.ds(rbase, 8)], dstb)
                pltpu.sync_copy(p_hbm.at[pl.ds(ebase, 1024)], pb)

                def micro(j, cc_):
                    pltpu.async_copy(h_hbm.at[srcb.at[j]], rows, sem).wait()

                    def scale(q, c3):
                        pbase = j * 128 + q * 16
                        for l in range(16):
                            bc = plsc.load_gather(
                                pb, [jnp.full((16,), pbase + l, I32)])
                            rows[q * 16 + l] = rows[q * 16 + l] * bc
                        return c3

                    lax.fori_loop(0, 8, scale, 0)
                    pltpu.sync_copy(rows, acc.at[dstb.at[j]], add=True)
                    return cc_

                lax.fori_loop(0, 8, micro, 0)
                return c_

            lax.fori_loop(0, n_chunks, chunk, 0)
            plsc.subcore_barrier()
            pltpu.sync_copy(
                acc.at[pl.ds(row0, rpt)],
                out_hbm.at[cid, c, pl.ds(row0, rpt)])

    return k(hflat, srcoff, dst2, p)


def _gat_layer(feats, w, att_s, att_d, bias, src, dst, dst2,
               n_pad, e_pad, e_real):
    k_in = feats.shape[1]
    d_out = w.shape[1]
    d_pad = _ceil_to(d_out + 1, 16)
    n_cc = d_pad // 16
    one_col = d_out

    w_pad = jnp.zeros((k_in, d_pad), F32).at[:w.shape[0], :d_out].set(w)
    s_pad = jnp.zeros((d_pad, 1), F32).at[:d_out, 0].set(att_s)
    dd_pad = jnp.zeros((d_pad, 1), F32).at[:d_out, 0].set(att_d)
    b_pad = jnp.zeros((1, d_pad), F32).at[0, :d_out].set(bias)

    h, a_s, a_d, bm = _tc_feats(feats, w_pad, s_pad, dd_pad, one_col, n_pad)
    cmax = jnp.maximum(jnp.max(bm[:, 0, 0]) + jnp.max(bm[:, 0, 1]), 0.0)
    cvec = jnp.full((16,), 1.0, F32) * cmax

    p = _sc_logits(a_s.reshape(n_pad), a_d.reshape(n_pad), src, dst, cvec,
                   n_pad, e_pad, e_real)

    coff = jnp.arange(n_cc, dtype=I32)[:, None]
    srcoff = (src[None, :] * n_cc + coff).reshape(n_cc, e_pad // 128, 128)
    accp = _sc_scatter(h.reshape(n_pad * n_cc, 16), srcoff, dst2, p,
                       n_pad, e_pad, n_cc)
    return _tc_norm(accp, b_pad, n_pad, d_pad, one_col), d_out


def kernel(x, edge_index, W1, att_src1, att_dst1, b1,
           W2, att_src2, att_dst2, b2):
    n, d_in = x.shape
    e = edge_index.shape[1]
    e_real = e + n
    n_pad = _ceil_to(n, _RB)
    e_pad = _ceil_to(e_real, 1024 * _NC * _NS)
    k_in = _ceil_to(d_in, 8)

    x_pad = jnp.zeros((n_pad, k_in), F32).at[:n, :d_in].set(x)
    loop = jnp.arange(n, dtype=I32)
    zpad = jnp.zeros((e_pad - e_real,), I32)
    src = jnp.concatenate([edge_index[0].astype(I32), loop, zpad])
    dst = jnp.concatenate([edge_index[1].astype(I32), loop, zpad])
    dst2 = dst.reshape(e_pad // 128, 128)

    h1, _ = _gat_layer(x_pad, W1, att_src1, att_dst1, b1, src, dst, dst2,
                       n_pad, e_pad, e_real)
    h2, d2 = _gat_layer(h1, W2, att_src2, att_dst2, b2, src, dst, dst2,
                        n_pad, e_pad, e_real)
    return (h2[:n, :d2], edge_index)


# speedup vs baseline: 17.5946x; 1.4937x over previous
import functools

import jax
import jax.numpy as jnp
from jax import lax
from jax.experimental import pallas as pl
from jax.experimental.pallas import tpu as pltpu
from jax.experimental.pallas import tpu_sc as plsc

F32 = jnp.float32
I32 = jnp.int32

_BCAST_DNUMS = lax.GatherDimensionNumbers(
    offset_dims=(), collapsed_slice_dims=(0,), start_index_map=(0,))


def _splat(vec, lane):
    idx = jnp.full((16, 1), lane, I32)
    return lax.gather(vec, idx, _BCAST_DNUMS, (1,),
                      mode=lax.GatherScatterMode.PROMISE_IN_BOUNDS)

_NC = 2
_NS = 16
_RB = 1024


def _ceil_to(v, m):
    return (v + m - 1) // m * m


def _tc_feats(x_in, w_pad, att_s, att_d, one_col, n_pad):
    k_in = x_in.shape[1]
    d_pad = w_pad.shape[1]
    grid = n_pad // _RB

    def body(x_ref, w_ref, s_ref, d_ref, h_ref, as_ref, ad_ref, bm_ref):
        h = jnp.dot(x_ref[...], w_ref[...], preferred_element_type=F32)
        col = lax.broadcasted_iota(I32, (_RB, d_pad), 1)
        h = h + (col == one_col).astype(F32)
        h_ref[...] = h
        a_s = jnp.dot(h, s_ref[...], preferred_element_type=F32)
        a_d = jnp.dot(h, d_ref[...], preferred_element_type=F32)
        as_ref[...] = a_s
        ad_ref[...] = a_d
        i0 = lax.broadcasted_iota(I32, (1, 1, 128), 2)
        bm_ref[...] = jnp.where(i0 == 0, jnp.max(a_s),
                                jnp.where(i0 == 1, jnp.max(a_d), -1e30))

    return pl.pallas_call(
        body,
        grid=(grid,),
        in_specs=[pl.BlockSpec((_RB, k_in), lambda i: (i, 0)),
                  pl.BlockSpec((k_in, d_pad), lambda i: (0, 0)),
                  pl.BlockSpec((d_pad, 1), lambda i: (0, 0)),
                  pl.BlockSpec((d_pad, 1), lambda i: (0, 0))],
        out_specs=[pl.BlockSpec((_RB, d_pad), lambda i: (i, 0)),
                   pl.BlockSpec((_RB, 1), lambda i: (i, 0)),
                   pl.BlockSpec((_RB, 1), lambda i: (i, 0)),
                   pl.BlockSpec((1, 1, 128), lambda i: (i, 0, 0))],
        out_shape=[jax.ShapeDtypeStruct((n_pad, d_pad), F32),
                   jax.ShapeDtypeStruct((n_pad, 1), F32),
                   jax.ShapeDtypeStruct((n_pad, 1), F32),
                   jax.ShapeDtypeStruct((grid, 1, 128), F32)],
    )(x_in, w_pad, att_s, att_d)


def _tc_norm(accp, bias_pad, n_pad, d_pad, one_col):
    grid = n_pad // _RB

    n_cc = d_pad // 16

    def body(a_ref, b_ref, o_ref):
        accc = a_ref[0] + a_ref[1]
        acc = jnp.concatenate([accc[c] for c in range(n_cc)], axis=-1)
        col = lax.broadcasted_iota(I32, (_RB, d_pad), 1)
        s = jnp.sum(jnp.where(col == one_col, acc, 0.0), axis=1, keepdims=True)
        o_ref[...] = jnp.maximum(acc / (s + 1e-16) + b_ref[...], 0.0)

    return pl.pallas_call(
        body,
        grid=(grid,),
        in_specs=[pl.BlockSpec((2, n_cc, _RB, 16), lambda i: (0, 0, i, 0)),
                  pl.BlockSpec((1, d_pad), lambda i: (0, 0))],
        out_specs=pl.BlockSpec((_RB, d_pad), lambda i: (i, 0)),
        out_shape=jax.ShapeDtypeStruct((n_pad, d_pad), F32),
    )(accp, bias_pad)


def _sc_logits(a_src, a_dst, src, dst, cvec, n_pad, e_pad, e_real):
    t_per = e_pad // (_NC * _NS)
    n_chunks = t_per // 1024
    mesh = plsc.VectorSubcoreMesh(core_axis_name="c", subcore_axis_name="s",
                                  num_cores=_NC, num_subcores=_NS)

    @functools.partial(
        pl.kernel,
        out_type=jax.ShapeDtypeStruct((e_pad,), F32),
        mesh=mesh,
        compiler_params=pltpu.CompilerParams(needs_layout_passes=False),
        scratch_types=[pltpu.VMEM((n_pad,), F32),
                       pltpu.VMEM((n_pad,), F32),
                       pltpu.VMEM((1024,), I32),
                       pltpu.VMEM((1024,), I32),
                       pltpu.VMEM((1024,), F32),
                       pltpu.VMEM((16,), F32)])
    def k(as_hbm, ad_hbm, src_hbm, dst_hbm, cv_hbm, p_hbm,
          as_v, ad_v, src_v, dst_v, p_v, cv_v):
        cid = lax.axis_index("c")
        sid = lax.axis_index("s")
        wid = cid * _NS + sid
        pltpu.sync_copy(as_hbm, as_v)
        pltpu.sync_copy(ad_hbm, ad_v)
        pltpu.sync_copy(cv_hbm, cv_v)
        cvr = cv_v[...]
        tbase = wid * t_per

        def chunk(g, carry):
            base = tbase + g * 1024
            pltpu.sync_copy(src_hbm.at[pl.ds(base, 1024)], src_v)
            pltpu.sync_copy(dst_hbm.at[pl.ds(base, 1024)], dst_v)

            def grp(i, c2):
                sv = src_v[pl.ds(i * 16, 16)]
                dv = dst_v[pl.ds(i * 16, 16)]
                e = plsc.load_gather(as_v, [sv]) + plsc.load_gather(ad_v, [dv])
                e = jnp.where(e > 0, e, 0.2 * e)
                gi = base + i * 16 + lax.iota(I32, 16)
                p_v[pl.ds(i * 16, 16)] = jnp.where(
                    gi < e_real, jnp.exp(e - cvr), 0.0)
                return c2

            lax.fori_loop(0, 64, grp, 0)
            pltpu.sync_copy(p_v, p_hbm.at[pl.ds(base, 1024)])
            return carry

        lax.fori_loop(0, n_chunks, chunk, 0)

    return k(a_src, a_dst, src, dst, cvec)


def _sc_scatter(hflat, srcoff, dst2, p, n_pad, e_pad, n_cc):
    t_per = e_pad // (_NC * _NS)
    n_chunks = t_per // 1024
    rpt = n_pad // _NS
    zr = 112
    nz = rpt // zr
    mesh = plsc.VectorSubcoreMesh(core_axis_name="c", subcore_axis_name="s",
                                  num_cores=_NC, num_subcores=_NS)

    @functools.partial(
        pl.kernel,
        out_type=jax.ShapeDtypeStruct((_NC, n_cc, n_pad, 16), F32),
        mesh=mesh,
        compiler_params=pltpu.CompilerParams(needs_layout_passes=False,
                                             use_tc_tiling_on_sc=False),
        scratch_types=[pltpu.VMEM((8, 128), I32),
                       pltpu.VMEM((8, 128), I32),
                       pltpu.VMEM((1024,), F32),
                       pltpu.VMEM((128, 16), F32),
                       pltpu.VMEM((128, 16), F32),
                       pltpu.VMEM((128, 16), F32),
                       pltpu.VMEM((zr, 16), F32),
                       pltpu.VMEM_SHARED((n_pad, 16), F32),
                       pltpu.SemaphoreType.DMA,
                       pltpu.SemaphoreType.DMA,
                       pltpu.SemaphoreType.DMA,
                       pltpu.SemaphoreType.DMA,
                       pltpu.SemaphoreType.DMA,
                       pltpu.SemaphoreType.DMA])
    def k(h_hbm, so_hbm, d2_hbm, p_hbm, out_hbm,
          srcb, dstb, pb, rows0, rows1, rows2, zbuf, acc,
          gs0, gs1, gs2, ss0, ss1, ss2):
        rows = [rows0, rows1, rows2]
        gsem = [gs0, gs1, gs2]
        ssem = [ss0, ss1, ss2]
        cid = lax.axis_index("c")
        sid = lax.axis_index("s")
        wid = cid * _NS + sid
        row0 = sid * rpt

        def zrow(r, c_):
            zbuf[r] = jnp.zeros((16,), F32)
            return c_

        lax.fori_loop(0, zr, zrow, 0)

        for c in range(n_cc):
            plsc.subcore_barrier()

            def zcp(z, c_):
                pltpu.sync_copy(zbuf, acc.at[pl.ds(row0 + z * zr, zr)])
                return c_

            lax.fori_loop(0, nz, zcp, 0)
            plsc.subcore_barrier()

            def chunk(g, c_):
                ebase = wid * t_per + g * 1024
                rbase = wid * (t_per // 128) + g * 8
                pltpu.sync_copy(so_hbm.at[c, pl.ds(rbase, 8)], srcb)
                pltpu.sync_copy(d2_hbm.at[pl.ds(rbase, 8)], dstb)
                pltpu.sync_copy(p_hbm.at[pl.ds(ebase, 1024)], pb)

                sdesc = [None, None, None]
                gdesc = [None, None, None]
                gdesc[0] = pltpu.async_copy(
                    h_hbm.at[srcb.at[0]], rows[0], gsem[0])
                for j in range(8):
                    b = j % 3
                    gdesc[b].wait()
                    if j + 1 < 8:
                        nb = (j + 1) % 3
                        if sdesc[nb] is not None:
                            sdesc[nb].wait()
                        gdesc[nb] = pltpu.async_copy(
                            h_hbm.at[srcb.at[j + 1]], rows[nb], gsem[nb])

                    def scale(q, c3, j=j, b=b):
                        pv = pb[pl.ds(j * 128 + q * 16, 16)]
                        for l in range(16):
                            r = q * 16 + l
                            rows[b][r] = rows[b][r] * _splat(pv, l)
                        return c3

                    lax.fori_loop(0, 8, scale, 0)
                    sdesc[b] = pltpu.async_copy(
                        rows[b], acc.at[dstb.at[j]], ssem[b], add=True)
                for b in range(3):
                    if sdesc[b] is not None:
                        sdesc[b].wait()
                return c_

            lax.fori_loop(0, n_chunks, chunk, 0)
            plsc.subcore_barrier()
            pltpu.sync_copy(
                acc.at[pl.ds(row0, rpt)],
                out_hbm.at[cid, c, pl.ds(row0, rpt)])

    return k(hflat, srcoff, dst2, p)


def _gat_layer(feats, w, att_s, att_d, bias, src, dst, dst2,
               n_pad, e_pad, e_real):
    k_in = feats.shape[1]
    d_out = w.shape[1]
    d_pad = _ceil_to(d_out + 1, 16)
    n_cc = d_pad // 16
    one_col = d_out

    w_pad = jnp.zeros((k_in, d_pad), F32).at[:w.shape[0], :d_out].set(w)
    s_pad = jnp.zeros((d_pad, 1), F32).at[:d_out, 0].set(att_s)
    dd_pad = jnp.zeros((d_pad, 1), F32).at[:d_out, 0].set(att_d)
    b_pad = jnp.zeros((1, d_pad), F32).at[0, :d_out].set(bias)

    h, a_s, a_d, bm = _tc_feats(feats, w_pad, s_pad, dd_pad, one_col, n_pad)
    cmax = jnp.maximum(jnp.max(bm[:, 0, 0]) + jnp.max(bm[:, 0, 1]), 0.0)
    cvec = jnp.full((16,), 1.0, F32) * cmax

    p = _sc_logits(a_s.reshape(n_pad), a_d.reshape(n_pad), src, dst, cvec,
                   n_pad, e_pad, e_real)

    coff = jnp.arange(n_cc, dtype=I32)[:, None]
    srcoff = (src[None, :] * n_cc + coff).reshape(n_cc, e_pad // 128, 128)
    accp = _sc_scatter(h.reshape(n_pad * n_cc, 16), srcoff, dst2, p,
                       n_pad, e_pad, n_cc)
    return _tc_norm(accp, b_pad, n_pad, d_pad, one_col), d_out


def kernel(x, edge_index, W1, att_src1, att_dst1, b1,
           W2, att_src2, att_dst2, b2):
    n, d_in = x.shape
    e = edge_index.shape[1]
    e_real = e + n
    n_pad = _ceil_to(n, _RB)
    e_pad = _ceil_to(e_real, 1024 * _NC * _NS)
    k_in = _ceil_to(d_in, 8)

    x_pad = jnp.zeros((n_pad, k_in), F32).at[:n, :d_in].set(x)
    loop = jnp.arange(n, dtype=I32)
    zpad = jnp.zeros((e_pad - e_real,), I32)
    src = jnp.concatenate([edge_index[0].astype(I32), loop, zpad])
    dst = jnp.concatenate([edge_index[1].astype(I32), loop, zpad])
    dst2 = dst.reshape(e_pad // 128, 128)

    h1, _ = _gat_layer(x_pad, W1, att_src1, att_dst1, b1, src, dst, dst2,
                       n_pad, e_pad, e_real)
    h2, d2 = _gat_layer(h1, W2, att_src2, att_dst2, b2, src, dst, dst2,
                        n_pad, e_pad, e_real)
    return (h2[:n, :d2], edge_index)


# speedup vs baseline: 19.5059x; 1.1086x over previous
import functools

import jax
import jax.numpy as jnp
from jax import lax
from jax.experimental import pallas as pl
from jax.experimental.pallas import tpu as pltpu
from jax.experimental.pallas import tpu_sc as plsc

F32 = jnp.float32
I32 = jnp.int32

_BCAST_DNUMS = lax.GatherDimensionNumbers(
    offset_dims=(), collapsed_slice_dims=(0,), start_index_map=(0,))


def _splat(vec, lane):
    idx = jnp.full((16, 1), lane, I32)
    return lax.gather(vec, idx, _BCAST_DNUMS, (1,),
                      mode=lax.GatherScatterMode.PROMISE_IN_BOUNDS)

_NC = 2
_NS = 16
_RB = 1024


def _ceil_to(v, m):
    return (v + m - 1) // m * m


_DSC = 128


def _tc_feats(x_in, w_pad, att_s, att_d, one_col, n_pad):
    k_in = x_in.shape[1]
    d_pad = w_pad.shape[1]
    grid = n_pad // _RB

    def body(x_ref, w_ref, s_ref, d_ref, h_ref, as_ref, ad_ref, bm_ref):
        h = jnp.dot(x_ref[...], w_ref[...], preferred_element_type=F32)
        col = lax.broadcasted_iota(I32, (_RB, d_pad), 1)
        h = h + (col == one_col).astype(F32)
        h_ref[...] = h
        a_s = jnp.dot(h, s_ref[...], preferred_element_type=F32)
        a_d = jnp.dot(h, d_ref[...], preferred_element_type=F32)
        as_ref[...] = a_s
        ad_ref[...] = a_d
        i0 = lax.broadcasted_iota(I32, (1, 1, 128), 2)
        bm_ref[...] = jnp.where(i0 == 0, jnp.max(a_s),
                                jnp.where(i0 == 1, jnp.max(a_d), -1e30))

    return pl.pallas_call(
        body,
        grid=(grid,),
        in_specs=[pl.BlockSpec((_RB, k_in), lambda i: (i, 0)),
                  pl.BlockSpec((k_in, d_pad), lambda i: (0, 0)),
                  pl.BlockSpec((d_pad, 1), lambda i: (0, 0)),
                  pl.BlockSpec((d_pad, 1), lambda i: (0, 0))],
        out_specs=[pl.BlockSpec((_RB, d_pad), lambda i: (i, 0)),
                   pl.BlockSpec((_RB, 1), lambda i: (i, 0)),
                   pl.BlockSpec((_RB, 1), lambda i: (i, 0)),
                   pl.BlockSpec((1, 1, 128), lambda i: (i, 0, 0))],
        out_shape=[jax.ShapeDtypeStruct((n_pad, d_pad), F32),
                   jax.ShapeDtypeStruct((n_pad, 1), F32),
                   jax.ShapeDtypeStruct((n_pad, 1), F32),
                   jax.ShapeDtypeStruct((grid, 1, 128), F32)],
    )(x_in, w_pad, att_s, att_d)


def _tc_norm(accp, bias_pad, n_pad, one_col):
    grid = n_pad // _RB

    def body(a_ref, b_ref, o_ref):
        acc = a_ref[0] + a_ref[1]
        col = lax.broadcasted_iota(I32, (_RB, _DSC), 1)
        s = jnp.sum(jnp.where(col == one_col, acc, 0.0), axis=1, keepdims=True)
        out = jnp.maximum(acc / (s + 1e-16) + b_ref[...], 0.0)
        o_ref[...] = jnp.where(col <= one_col, out, 0.0)

    return pl.pallas_call(
        body,
        grid=(grid,),
        in_specs=[pl.BlockSpec((2, _RB, _DSC), lambda i: (0, i, 0)),
                  pl.BlockSpec((1, _DSC), lambda i: (0, 0))],
        out_specs=pl.BlockSpec((_RB, _DSC), lambda i: (i, 0)),
        out_shape=jax.ShapeDtypeStruct((n_pad, _DSC), F32),
    )(accp, bias_pad)


def _sc_logits(a_src, a_dst, src, dst, cvec, n_pad, e_pad, e_real):
    t_per = e_pad // (_NC * _NS)
    n_chunks = t_per // 1024
    mesh = plsc.VectorSubcoreMesh(core_axis_name="c", subcore_axis_name="s",
                                  num_cores=_NC, num_subcores=_NS)

    @functools.partial(
        pl.kernel,
        out_type=jax.ShapeDtypeStruct((e_pad,), F32),
        mesh=mesh,
        compiler_params=pltpu.CompilerParams(needs_layout_passes=False),
        scratch_types=[pltpu.VMEM((n_pad,), F32),
                       pltpu.VMEM((n_pad,), F32),
                       pltpu.VMEM((1024,), I32),
                       pltpu.VMEM((1024,), I32),
                       pltpu.VMEM((1024,), F32),
                       pltpu.VMEM((16,), F32)])
    def k(as_hbm, ad_hbm, src_hbm, dst_hbm, cv_hbm, p_hbm,
          as_v, ad_v, src_v, dst_v, p_v, cv_v):
        cid = lax.axis_index("c")
        sid = lax.axis_index("s")
        wid = cid * _NS + sid
        pltpu.sync_copy(as_hbm, as_v)
        pltpu.sync_copy(ad_hbm, ad_v)
        pltpu.sync_copy(cv_hbm, cv_v)
        cvr = cv_v[...]
        tbase = wid * t_per

        def chunk(g, carry):
            base = tbase + g * 1024
            pltpu.sync_copy(src_hbm.at[pl.ds(base, 1024)], src_v)
            pltpu.sync_copy(dst_hbm.at[pl.ds(base, 1024)], dst_v)

            def grp(i, c2):
                sv = src_v[pl.ds(i * 16, 16)]
                dv = dst_v[pl.ds(i * 16, 16)]
                e = plsc.load_gather(as_v, [sv]) + plsc.load_gather(ad_v, [dv])
                e = jnp.where(e > 0, e, 0.2 * e)
                gi = base + i * 16 + lax.iota(I32, 16)
                p_v[pl.ds(i * 16, 16)] = jnp.where(
                    gi < e_real, jnp.exp(e - cvr), 0.0)
                return c2

            lax.fori_loop(0, 64, grp, 0)
            pltpu.sync_copy(p_v, p_hbm.at[pl.ds(base, 1024)])
            return carry

        lax.fori_loop(0, n_chunks, chunk, 0)

    return k(a_src, a_dst, src, dst, cvec)


def _sc_scatter(hflat, srcoff, dst2, p, n_pad, e_pad, n_cc):
    t_per = e_pad // (_NC * _NS)
    n_chunks = t_per // 1024
    rpt = n_pad // _NS
    zr = 112
    nz = rpt // zr
    mesh = plsc.VectorSubcoreMesh(core_axis_name="c", subcore_axis_name="s",
                                  num_cores=_NC, num_subcores=_NS)

    @functools.partial(
        pl.kernel,
        out_type=jax.ShapeDtypeStruct((_NC, n_pad, _DSC), F32),
        mesh=mesh,
        compiler_params=pltpu.CompilerParams(needs_layout_passes=False,
                                             use_tc_tiling_on_sc=False),
        scratch_types=[pltpu.VMEM((8, 128), I32),
                       pltpu.VMEM((8, 128), I32),
                       pltpu.VMEM((1024,), F32),
                       pltpu.VMEM((128, 16), F32),
                       pltpu.VMEM((128, 16), F32),
                       pltpu.VMEM((128, 16), F32),
                       pltpu.VMEM((zr, 16), F32),
                       pltpu.VMEM_SHARED((n_pad, 16), F32),
                       pltpu.SemaphoreType.DMA,
                       pltpu.SemaphoreType.DMA,
                       pltpu.SemaphoreType.DMA,
                       pltpu.SemaphoreType.DMA,
                       pltpu.SemaphoreType.DMA,
                       pltpu.SemaphoreType.DMA])
    def k(h_hbm, so_hbm, d2_hbm, p_hbm, out_hbm,
          srcb, dstb, pb, rows0, rows1, rows2, zbuf, acc,
          gs0, gs1, gs2, ss0, ss1, ss2):
        rows = [rows0, rows1, rows2]
        gsem = [gs0, gs1, gs2]
        ssem = [ss0, ss1, ss2]
        cid = lax.axis_index("c")
        sid = lax.axis_index("s")
        wid = cid * _NS + sid
        row0 = sid * rpt

        def zrow(r, c_):
            zbuf[r] = jnp.zeros((16,), F32)
            return c_

        lax.fori_loop(0, zr, zrow, 0)

        for c in range(n_cc):
            plsc.subcore_barrier()

            def zcp(z, c_):
                pltpu.sync_copy(zbuf, acc.at[pl.ds(row0 + z * zr, zr)])
                return c_

            lax.fori_loop(0, nz, zcp, 0)
            plsc.subcore_barrier()

            def chunk(g, c_):
                ebase = wid * t_per + g * 1024
                rbase = wid * (t_per // 128) + g * 8
                pltpu.sync_copy(so_hbm.at[c, pl.ds(rbase, 8)], srcb)
                pltpu.sync_copy(d2_hbm.at[pl.ds(rbase, 8)], dstb)
                pltpu.sync_copy(p_hbm.at[pl.ds(ebase, 1024)], pb)

                sdesc = [None, None, None]
                gdesc = [None, None, None]
                gdesc[0] = pltpu.async_copy(
                    h_hbm.at[srcb.at[0]], rows[0], gsem[0])
                for j in range(8):
                    b = j % 3
                    gdesc[b].wait()
                    if j + 1 < 8:
                        nb = (j + 1) % 3
                        if sdesc[nb] is not None:
                            sdesc[nb].wait()
                        gdesc[nb] = pltpu.async_copy(
                            h_hbm.at[srcb.at[j + 1]], rows[nb], gsem[nb])

                    def scale(q, c3, j=j, b=b):
                        pv = pb[pl.ds(j * 128 + q * 16, 16)]
                        for l in range(16):
                            r = q * 16 + l
                            rows[b][r] = rows[b][r] * _splat(pv, l)
                        return c3

                    lax.fori_loop(0, 8, scale, 0)
                    sdesc[b] = pltpu.async_copy(
                        rows[b], acc.at[dstb.at[j]], ssem[b], add=True)
                for b in range(3):
                    if sdesc[b] is not None:
                        sdesc[b].wait()
                return c_

            lax.fori_loop(0, n_chunks, chunk, 0)
            plsc.subcore_barrier()
            pltpu.sync_copy(
                acc.at[pl.ds(row0, rpt)],
                out_hbm.at[cid, pl.ds(row0, rpt), pl.ds(c * 16, 16)])

    return k(hflat, srcoff, dst2, p)


def _gat_layer(feats, w, att_s, att_d, bias, src, dst, dst2, srcoff,
               n_pad, e_pad, e_real):
    k_in = feats.shape[1]
    d_out = w.shape[1]
    n_cc = _ceil_to(d_out + 1, 16) // 16
    one_col = d_out

    w_pad = jnp.zeros((k_in, _DSC), F32).at[:w.shape[0], :d_out].set(w)
    s_pad = jnp.zeros((_DSC, 1), F32).at[:d_out, 0].set(att_s)
    dd_pad = jnp.zeros((_DSC, 1), F32).at[:d_out, 0].set(att_d)
    b_pad = jnp.zeros((1, _DSC), F32).at[0, :d_out].set(bias)

    h, a_s, a_d, bm = _tc_feats(feats, w_pad, s_pad, dd_pad, one_col, n_pad)
    cmax = jnp.maximum(jnp.max(bm[:, 0, 0]) + jnp.max(bm[:, 0, 1]), 0.0)
    cvec = jnp.full((16,), 1.0, F32) * cmax

    p = _sc_logits(a_s.reshape(n_pad), a_d.reshape(n_pad), src, dst, cvec,
                   n_pad, e_pad, e_real)

    accp = _sc_scatter(h.reshape(n_pad * 8, 16), srcoff, dst2, p,
                       n_pad, e_pad, n_cc)
    return _tc_norm(accp, b_pad, n_pad, one_col), d_out


def kernel(x, edge_index, W1, att_src1, att_dst1, b1,
           W2, att_src2, att_dst2, b2):
    n, d_in = x.shape
    e = edge_index.shape[1]
    e_real = e + n
    n_pad = _ceil_to(n, _RB)
    e_pad = _ceil_to(e_real, 1024 * _NC * _NS)
    k_in = _ceil_to(d_in, 8)

    x_pad = jnp.zeros((n_pad, k_in), F32).at[:n, :d_in].set(x)
    loop = jnp.arange(n, dtype=I32)
    zpad = jnp.zeros((e_pad - e_real,), I32)
    src = jnp.concatenate([edge_index[0].astype(I32), loop, zpad])
    dst = jnp.concatenate([edge_index[1].astype(I32), loop, zpad])
    dst2 = dst.reshape(e_pad // 128, 128)
    coff = jnp.arange(7, dtype=I32)[:, None]
    srcoff = (src[None, :] * 8 + coff).reshape(7, e_pad // 128, 128)

    h1, _ = _gat_layer(x_pad, W1, att_src1, att_dst1, b1, src, dst, dst2,
                       srcoff, n_pad, e_pad, e_real)
    h2, d2 = _gat_layer(h1, W2, att_src2, att_dst2, b2, src, dst, dst2,
                        srcoff, n_pad, e_pad, e_real)
    return (h2[:n, :d2], edge_index)


# speedup vs baseline: 20.2460x; 1.0379x over previous
import functools

import jax
import jax.numpy as jnp
from jax import lax
from jax.experimental import pallas as pl
from jax.experimental.pallas import tpu as pltpu
from jax.experimental.pallas import tpu_sc as plsc

F32 = jnp.float32
I32 = jnp.int32

_BCAST_DNUMS = lax.GatherDimensionNumbers(
    offset_dims=(), collapsed_slice_dims=(0,), start_index_map=(0,))


def _splat(vec, lane):
    idx = jnp.full((16, 1), lane, I32)
    return lax.gather(vec, idx, _BCAST_DNUMS, (1,),
                      mode=lax.GatherScatterMode.PROMISE_IN_BOUNDS)

_NC = 2
_NS = 16
_RB = 1024


def _ceil_to(v, m):
    return (v + m - 1) // m * m


_DSC = 128


def _tc_feats(x_in, w_pad, att_s, att_d, one_col, n_pad):
    k_in = x_in.shape[1]
    d_pad = w_pad.shape[1]
    grid = n_pad // _RB

    def body(x_ref, w_ref, s_ref, d_ref, h_ref, as_ref, ad_ref, bm_ref):
        h = jnp.dot(x_ref[...], w_ref[...], preferred_element_type=F32)
        col = lax.broadcasted_iota(I32, (_RB, d_pad), 1)
        h = h + (col == one_col).astype(F32)
        h_ref[...] = h
        a_s = jnp.dot(h, s_ref[...], preferred_element_type=F32)
        a_d = jnp.dot(h, d_ref[...], preferred_element_type=F32)
        as_ref[...] = a_s
        ad_ref[...] = a_d
        i0 = lax.broadcasted_iota(I32, (1, 1, 128), 2)
        bm_ref[...] = jnp.where(i0 == 0, jnp.max(a_s),
                                jnp.where(i0 == 1, jnp.max(a_d), -1e30))

    return pl.pallas_call(
        body,
        grid=(grid,),
        in_specs=[pl.BlockSpec((_RB, k_in), lambda i: (i, 0)),
                  pl.BlockSpec((k_in, d_pad), lambda i: (0, 0)),
                  pl.BlockSpec((d_pad, 1), lambda i: (0, 0)),
                  pl.BlockSpec((d_pad, 1), lambda i: (0, 0))],
        out_specs=[pl.BlockSpec((_RB, d_pad), lambda i: (i, 0)),
                   pl.BlockSpec((_RB, 1), lambda i: (i, 0)),
                   pl.BlockSpec((_RB, 1), lambda i: (i, 0)),
                   pl.BlockSpec((1, 1, 128), lambda i: (i, 0, 0))],
        out_shape=[jax.ShapeDtypeStruct((n_pad, d_pad), F32),
                   jax.ShapeDtypeStruct((n_pad, 1), F32),
                   jax.ShapeDtypeStruct((n_pad, 1), F32),
                   jax.ShapeDtypeStruct((grid, 1, 128), F32)],
    )(x_in, w_pad, att_s, att_d)


def _tc_norm(accp, bias_pad, n_pad, one_col):
    grid = n_pad // _RB

    def body(a_ref, b_ref, o_ref):
        acc = a_ref[0] + a_ref[1]
        col = lax.broadcasted_iota(I32, (_RB, _DSC), 1)
        s = jnp.sum(jnp.where(col == one_col, acc, 0.0), axis=1, keepdims=True)
        out = jnp.maximum(acc / (s + 1e-16) + b_ref[...], 0.0)
        o_ref[...] = jnp.where(col <= one_col, out, 0.0)

    return pl.pallas_call(
        body,
        grid=(grid,),
        in_specs=[pl.BlockSpec((2, _RB, _DSC), lambda i: (0, i, 0)),
                  pl.BlockSpec((1, _DSC), lambda i: (0, 0))],
        out_specs=pl.BlockSpec((_RB, _DSC), lambda i: (i, 0)),
        out_shape=jax.ShapeDtypeStruct((n_pad, _DSC), F32),
    )(accp, bias_pad)


def _sc_logits(a_src, a_dst, src, dst, cvec, n_pad, e_pad, e_real):
    t_per = e_pad // (_NC * _NS)
    n_chunks = t_per // 1024
    mesh = plsc.VectorSubcoreMesh(core_axis_name="c", subcore_axis_name="s",
                                  num_cores=_NC, num_subcores=_NS)

    @functools.partial(
        pl.kernel,
        out_type=jax.ShapeDtypeStruct((e_pad,), F32),
        mesh=mesh,
        compiler_params=pltpu.CompilerParams(needs_layout_passes=False),
        scratch_types=[pltpu.VMEM((n_pad,), F32),
                       pltpu.VMEM((n_pad,), F32),
                       pltpu.VMEM((1024,), I32),
                       pltpu.VMEM((1024,), I32),
                       pltpu.VMEM((1024,), F32),
                       pltpu.VMEM((16,), F32)])
    def k(as_hbm, ad_hbm, src_hbm, dst_hbm, cv_hbm, p_hbm,
          as_v, ad_v, src_v, dst_v, p_v, cv_v):
        cid = lax.axis_index("c")
        sid = lax.axis_index("s")
        wid = cid * _NS + sid
        pltpu.sync_copy(as_hbm, as_v)
        pltpu.sync_copy(ad_hbm, ad_v)
        pltpu.sync_copy(cv_hbm, cv_v)
        cvr = cv_v[...]
        tbase = wid * t_per

        def chunk(g, carry):
            base = tbase + g * 1024
            pltpu.sync_copy(src_hbm.at[pl.ds(base, 1024)], src_v)
            pltpu.sync_copy(dst_hbm.at[pl.ds(base, 1024)], dst_v)

            def grp(i, c2):
                sv = src_v[pl.ds(i * 16, 16)]
                dv = dst_v[pl.ds(i * 16, 16)]
                e = plsc.load_gather(as_v, [sv]) + plsc.load_gather(ad_v, [dv])
                e = jnp.where(e > 0, e, 0.2 * e)
                gi = base + i * 16 + lax.iota(I32, 16)
                p_v[pl.ds(i * 16, 16)] = jnp.where(
                    gi < e_real, jnp.exp(e - cvr), 0.0)
                return c2

            lax.fori_loop(0, 64, grp, 0)
            pltpu.sync_copy(p_v, p_hbm.at[pl.ds(base, 1024)])
            return carry

        lax.fori_loop(0, n_chunks, chunk, 0)

    return k(a_src, a_dst, src, dst, cvec)


def _sc_scatter(hflat, srcoff, dst2, p, n_pad, e_pad, n_cc):
    t_per = e_pad // (_NC * _NS)
    n_chunks = t_per // 1024
    rpt = n_pad // _NS
    zr = 112
    nz = rpt // zr
    mesh = plsc.VectorSubcoreMesh(core_axis_name="c", subcore_axis_name="s",
                                  num_cores=_NC, num_subcores=_NS)

    @functools.partial(
        pl.kernel,
        out_type=jax.ShapeDtypeStruct((_NC, n_pad, _DSC), F32),
        mesh=mesh,
        compiler_params=pltpu.CompilerParams(needs_layout_passes=False,
                                             use_tc_tiling_on_sc=False),
        scratch_types=[pltpu.VMEM((8, 128), I32),
                       pltpu.VMEM((8, 128), I32),
                       pltpu.VMEM((1024,), F32),
                       pltpu.VMEM((128, 16), F32),
                       pltpu.VMEM((128, 16), F32),
                       pltpu.VMEM((128, 16), F32),
                       pltpu.VMEM((zr, 16), F32),
                       pltpu.VMEM_SHARED((n_pad, 16), F32),
                       pltpu.VMEM_SHARED((n_pad, 16), F32),
                       pltpu.SemaphoreType.DMA,
                       pltpu.SemaphoreType.DMA,
                       pltpu.SemaphoreType.DMA,
                       pltpu.SemaphoreType.DMA,
                       pltpu.SemaphoreType.DMA,
                       pltpu.SemaphoreType.DMA,
                       pltpu.SemaphoreType.DMA,
                       pltpu.SemaphoreType.DMA])
    def k(h_hbm, so_hbm, d2_hbm, p_hbm, out_hbm,
          srcb, dstb, pb, rows0, rows1, rows2, zbuf, acc0, acc1,
          gs0, gs1, gs2, ss0, ss1, ss2, os0, os1):
        rows = [rows0, rows1, rows2]
        gsem = [gs0, gs1, gs2]
        ssem = [ss0, ss1, ss2]
        accs = [acc0, acc1]
        osem = [os0, os1]
        cid = lax.axis_index("c")
        sid = lax.axis_index("s")
        wid = cid * _NS + sid
        row0 = sid * rpt

        def zrow(r, c_):
            zbuf[r] = jnp.zeros((16,), F32)
            return c_

        lax.fori_loop(0, zr, zrow, 0)

        odesc = [None, None]
        for c in range(n_cc):
            par = c % 2
            acc = accs[par]
            if odesc[par] is not None:
                odesc[par].wait()

            def zcp(z, c_, acc=acc):
                pltpu.sync_copy(zbuf, acc.at[pl.ds(row0 + z * zr, zr)])
                return c_

            lax.fori_loop(0, nz, zcp, 0)
            plsc.subcore_barrier()

            def chunk(g, c_):
                ebase = wid * t_per + g * 1024
                rbase = wid * (t_per // 128) + g * 8
                pltpu.sync_copy(so_hbm.at[c, pl.ds(rbase, 8)], srcb)
                pltpu.sync_copy(d2_hbm.at[pl.ds(rbase, 8)], dstb)
                pltpu.sync_copy(p_hbm.at[pl.ds(ebase, 1024)], pb)

                sdesc = [None, None, None]
                gdesc = [None, None, None]
                gdesc[0] = pltpu.async_copy(
                    h_hbm.at[srcb.at[0]], rows[0], gsem[0])
                for j in range(8):
                    b = j % 3
                    gdesc[b].wait()
                    if j + 1 < 8:
                        nb = (j + 1) % 3
                        if sdesc[nb] is not None:
                            sdesc[nb].wait()
                        gdesc[nb] = pltpu.async_copy(
                            h_hbm.at[srcb.at[j + 1]], rows[nb], gsem[nb])

                    def scale(q, c3, j=j, b=b):
                        pv = pb[pl.ds(j * 128 + q * 16, 16)]
                        for l in range(16):
                            r = q * 16 + l
                            rows[b][r] = rows[b][r] * _splat(pv, l)
                        return c3

                    lax.fori_loop(0, 8, scale, 0)
                    sdesc[b] = pltpu.async_copy(
                        rows[b], acc.at[dstb.at[j]], ssem[b], add=True)
                for b in range(3):
                    if sdesc[b] is not None:
                        sdesc[b].wait()
                return c_

            lax.fori_loop(0, n_chunks, chunk, 0)
            plsc.subcore_barrier()
            odesc[par] = pltpu.async_copy(
                acc.at[pl.ds(row0, rpt)],
                out_hbm.at[cid, pl.ds(row0, rpt), pl.ds(c * 16, 16)],
                osem[par])
        for par in range(2):
            if odesc[par] is not None:
                odesc[par].wait()

    return k(hflat, srcoff, dst2, p)


def _gat_layer(feats, w, att_s, att_d, bias, src, dst, dst2, srcoff,
               n_pad, e_pad, e_real):
    k_in = feats.shape[1]
    d_out = w.shape[1]
    n_cc = _ceil_to(d_out + 1, 16) // 16
    one_col = d_out

    w_pad = jnp.zeros((k_in, _DSC), F32).at[:w.shape[0], :d_out].set(w)
    s_pad = jnp.zeros((_DSC, 1), F32).at[:d_out, 0].set(att_s)
    dd_pad = jnp.zeros((_DSC, 1), F32).at[:d_out, 0].set(att_d)
    b_pad = jnp.zeros((1, _DSC), F32).at[0, :d_out].set(bias)

    h, a_s, a_d, bm = _tc_feats(feats, w_pad, s_pad, dd_pad, one_col, n_pad)
    cmax = jnp.maximum(jnp.max(bm[:, 0, 0]) + jnp.max(bm[:, 0, 1]), 0.0)
    cvec = jnp.full((16,), 1.0, F32) * cmax

    p = _sc_logits(a_s.reshape(n_pad), a_d.reshape(n_pad), src, dst, cvec,
                   n_pad, e_pad, e_real)

    accp = _sc_scatter(h.reshape(n_pad * 8, 16), srcoff, dst2, p,
                       n_pad, e_pad, n_cc)
    return _tc_norm(accp, b_pad, n_pad, one_col), d_out


def kernel(x, edge_index, W1, att_src1, att_dst1, b1,
           W2, att_src2, att_dst2, b2):
    n, d_in = x.shape
    e = edge_index.shape[1]
    e_real = e + n
    n_pad = _ceil_to(n, _RB)
    e_pad = _ceil_to(e_real, 1024 * _NC * _NS)
    k_in = _ceil_to(d_in, 8)

    x_pad = jnp.zeros((n_pad, k_in), F32).at[:n, :d_in].set(x)
    loop = jnp.arange(n, dtype=I32)
    zpad = jnp.zeros((e_pad - e_real,), I32)
    src = jnp.concatenate([edge_index[0].astype(I32), loop, zpad])
    dst = jnp.concatenate([edge_index[1].astype(I32), loop, zpad])
    dst2 = dst.reshape(e_pad // 128, 128)
    coff = jnp.arange(7, dtype=I32)[:, None]
    srcoff = (src[None, :] * 8 + coff).reshape(7, e_pad // 128, 128)

    h1, _ = _gat_layer(x_pad, W1, att_src1, att_dst1, b1, src, dst, dst2,
                       srcoff, n_pad, e_pad, e_real)
    h2, d2 = _gat_layer(h1, W2, att_src2, att_dst2, b2, src, dst, dst2,
                        srcoff, n_pad, e_pad, e_real)
    return (h2[:n, :d2], edge_index)


# speedup vs baseline: 21.5586x; 1.0648x over previous
import functools

import jax
import jax.numpy as jnp
from jax import lax
from jax.experimental import pallas as pl
from jax.experimental.pallas import tpu as pltpu
from jax.experimental.pallas import tpu_sc as plsc

F32 = jnp.float32
I32 = jnp.int32

_BCAST_DNUMS = lax.GatherDimensionNumbers(
    offset_dims=(), collapsed_slice_dims=(0,), start_index_map=(0,))


def _splat(vec, lane):
    idx = jnp.full((16, 1), lane, I32)
    return lax.gather(vec, idx, _BCAST_DNUMS, (1,),
                      mode=lax.GatherScatterMode.PROMISE_IN_BOUNDS)

_NC = 2
_NS = 16
_RB = 1024


def _ceil_to(v, m):
    return (v + m - 1) // m * m


_DSC = 128


def _tc_feats(x_in, w_pad, att_s, att_d, one_col, n_pad):
    k_in = x_in.shape[1]
    d_pad = w_pad.shape[1]
    grid = n_pad // _RB

    def body(x_ref, w_ref, s_ref, d_ref, h_ref, as_ref, ad_ref, bm_ref):
        h = jnp.dot(x_ref[...], w_ref[...], preferred_element_type=F32)
        col = lax.broadcasted_iota(I32, (_RB, d_pad), 1)
        h = h + (col == one_col).astype(F32)
        h_ref[...] = h
        a_s = jnp.dot(h, s_ref[...], preferred_element_type=F32)
        a_d = jnp.dot(h, d_ref[...], preferred_element_type=F32)
        as_ref[...] = a_s
        ad_ref[...] = a_d
        i0 = lax.broadcasted_iota(I32, (1, 1, 128), 2)
        bm_ref[...] = jnp.where(i0 == 0, jnp.max(a_s),
                                jnp.where(i0 == 1, jnp.max(a_d), -1e30))

    return pl.pallas_call(
        body,
        grid=(grid,),
        in_specs=[pl.BlockSpec((_RB, k_in), lambda i: (i, 0)),
                  pl.BlockSpec((k_in, d_pad), lambda i: (0, 0)),
                  pl.BlockSpec((d_pad, 1), lambda i: (0, 0)),
                  pl.BlockSpec((d_pad, 1), lambda i: (0, 0))],
        out_specs=[pl.BlockSpec((_RB, d_pad), lambda i: (i, 0)),
                   pl.BlockSpec((_RB, 1), lambda i: (i, 0)),
                   pl.BlockSpec((_RB, 1), lambda i: (i, 0)),
                   pl.BlockSpec((1, 1, 128), lambda i: (i, 0, 0))],
        out_shape=[jax.ShapeDtypeStruct((n_pad, d_pad), F32),
                   jax.ShapeDtypeStruct((n_pad, 1), F32),
                   jax.ShapeDtypeStruct((n_pad, 1), F32),
                   jax.ShapeDtypeStruct((grid, 1, 128), F32)],
    )(x_in, w_pad, att_s, att_d)


def _tc_norm(accp, bias_pad, n_pad, one_col):
    grid = n_pad // _RB

    def body(a_ref, b_ref, o_ref):
        acc = a_ref[0] + a_ref[1]
        col = lax.broadcasted_iota(I32, (_RB, _DSC), 1)
        s = jnp.sum(jnp.where(col == one_col, acc, 0.0), axis=1, keepdims=True)
        out = jnp.maximum(acc / (s + 1e-16) + b_ref[...], 0.0)
        o_ref[...] = jnp.where(col <= one_col, out, 0.0)

    return pl.pallas_call(
        body,
        grid=(grid,),
        in_specs=[pl.BlockSpec((2, _RB, _DSC), lambda i: (0, i, 0)),
                  pl.BlockSpec((1, _DSC), lambda i: (0, 0))],
        out_specs=pl.BlockSpec((_RB, _DSC), lambda i: (i, 0)),
        out_shape=jax.ShapeDtypeStruct((n_pad, _DSC), F32),
    )(accp, bias_pad)


def _sc_logits(a_src, a_dst, src, dst, cvec, n_pad, e_pad, e_real):
    t_per = e_pad // (_NC * _NS)
    n_chunks = t_per // 1024
    mesh = plsc.VectorSubcoreMesh(core_axis_name="c", subcore_axis_name="s",
                                  num_cores=_NC, num_subcores=_NS)

    @functools.partial(
        pl.kernel,
        out_type=jax.ShapeDtypeStruct((e_pad,), F32),
        mesh=mesh,
        compiler_params=pltpu.CompilerParams(needs_layout_passes=False),
        scratch_types=[pltpu.VMEM((n_pad,), F32),
                       pltpu.VMEM((n_pad,), F32),
                       pltpu.VMEM((1024,), I32),
                       pltpu.VMEM((1024,), I32),
                       pltpu.VMEM((1024,), F32),
                       pltpu.VMEM((16,), F32)])
    def k(as_hbm, ad_hbm, src_hbm, dst_hbm, cv_hbm, p_hbm,
          as_v, ad_v, src_v, dst_v, p_v, cv_v):
        cid = lax.axis_index("c")
        sid = lax.axis_index("s")
        wid = cid * _NS + sid
        pltpu.sync_copy(as_hbm, as_v)
        pltpu.sync_copy(ad_hbm, ad_v)
        pltpu.sync_copy(cv_hbm, cv_v)
        cvr = cv_v[...]
        tbase = wid * t_per

        def chunk(g, carry):
            base = tbase + g * 1024
            pltpu.sync_copy(src_hbm.at[pl.ds(base, 1024)], src_v)
            pltpu.sync_copy(dst_hbm.at[pl.ds(base, 1024)], dst_v)

            def grp(i, c2):
                sv = src_v[pl.ds(i * 16, 16)]
                dv = dst_v[pl.ds(i * 16, 16)]
                e = plsc.load_gather(as_v, [sv]) + plsc.load_gather(ad_v, [dv])
                e = jnp.where(e > 0, e, 0.2 * e)
                gi = base + i * 16 + lax.iota(I32, 16)
                p_v[pl.ds(i * 16, 16)] = jnp.where(
                    gi < e_real, jnp.exp(e - cvr), 0.0)
                return c2

            lax.fori_loop(0, 64, grp, 0)
            pltpu.sync_copy(p_v, p_hbm.at[pl.ds(base, 1024)])
            return carry

        lax.fori_loop(0, n_chunks, chunk, 0)

    return k(a_src, a_dst, src, dst, cvec)


def _sc_scatter(hflat, srcoff, dst2, p, n_pad, e_pad, n_cc):
    t_per = e_pad // (_NC * _NS)
    n_chunks = t_per // 1024
    rpt = n_pad // _NS
    zr = 112
    nz = rpt // zr
    mesh = plsc.VectorSubcoreMesh(core_axis_name="c", subcore_axis_name="s",
                                  num_cores=_NC, num_subcores=_NS)

    @functools.partial(
        pl.kernel,
        out_type=jax.ShapeDtypeStruct((_NC, n_pad, _DSC), F32),
        mesh=mesh,
        compiler_params=pltpu.CompilerParams(needs_layout_passes=False,
                                             use_tc_tiling_on_sc=False),
        scratch_types=[pltpu.VMEM((8, 128), I32),
                       pltpu.VMEM((t_per // 128, 128), I32),
                       pltpu.VMEM((t_per,), F32),
                       pltpu.VMEM((128, 16), F32),
                       pltpu.VMEM((128, 16), F32),
                       pltpu.VMEM((128, 16), F32),
                       pltpu.VMEM((zr, 16), F32),
                       pltpu.VMEM_SHARED((n_pad, 16), F32),
                       pltpu.SemaphoreType.DMA,
                       pltpu.SemaphoreType.DMA,
                       pltpu.SemaphoreType.DMA,
                       pltpu.SemaphoreType.DMA,
                       pltpu.SemaphoreType.DMA,
                       pltpu.SemaphoreType.DMA])
    def k(h_hbm, so_hbm, d2_hbm, p_hbm, out_hbm,
          srcb, dstb, pb, rows0, rows1, rows2, zbuf, acc,
          gs0, gs1, gs2, ss0, ss1, ss2):
        rows = [rows0, rows1, rows2]
        gsem = [gs0, gs1, gs2]
        ssem = [ss0, ss1, ss2]
        cid = lax.axis_index("c")
        sid = lax.axis_index("s")
        wid = cid * _NS + sid
        row0 = sid * rpt
        irows = t_per // 128

        def zrow(r, c_):
            zbuf[r] = jnp.zeros((16,), F32)
            return c_

        lax.fori_loop(0, zr, zrow, 0)
        pltpu.sync_copy(d2_hbm.at[pl.ds(wid * irows, irows)], dstb)
        pltpu.sync_copy(p_hbm.at[pl.ds(wid * t_per, t_per)], pb)

        for c in range(n_cc):
            def zcp(z, c_):
                pltpu.sync_copy(zbuf, acc.at[pl.ds(row0 + z * zr, zr)])
                return c_

            lax.fori_loop(0, nz, zcp, 0)
            plsc.subcore_barrier()

            def chunk(g, c_):
                pltpu.sync_copy(
                    so_hbm.at[c, pl.ds(wid * (t_per // 128) + g * 8, 8)],
                    srcb)
                sdesc = [None, None, None]
                gdesc = [None, None, None]
                gdesc[0] = pltpu.async_copy(
                    h_hbm.at[srcb.at[0]], rows[0], gsem[0])
                for j in range(8):
                    b = j % 3
                    gdesc[b].wait()
                    if j + 1 < 8:
                        nb = (j + 1) % 3
                        if sdesc[nb] is not None:
                            sdesc[nb].wait()
                        gdesc[nb] = pltpu.async_copy(
                            h_hbm.at[srcb.at[j + 1]], rows[nb], gsem[nb])

                    def scale(q, c3, j=j, b=b):
                        pv = pb[pl.ds(g * 1024 + j * 128 + q * 16, 16)]
                        for l in range(16):
                            r = q * 16 + l
                            rows[b][r] = rows[b][r] * _splat(pv, l)
                        return c3

                    lax.fori_loop(0, 8, scale, 0)
                    sdesc[b] = pltpu.async_copy(
                        rows[b], acc.at[dstb.at[g * 8 + j]], ssem[b],
                        add=True)
                for b in range(3):
                    if sdesc[b] is not None:
                        sdesc[b].wait()
                return c_

            lax.fori_loop(0, n_chunks, chunk, 0)
            plsc.subcore_barrier()
            pltpu.sync_copy(
                acc.at[pl.ds(row0, rpt)],
                out_hbm.at[cid, pl.ds(row0, rpt), pl.ds(c * 16, 16)])

    return k(hflat, srcoff, dst2, p)


def _gat_layer(feats, w, att_s, att_d, bias, src, dst, dst2, srcoff,
               n_pad, e_pad, e_real):
    k_in = feats.shape[1]
    d_out = w.shape[1]
    n_cc = _ceil_to(d_out + 1, 16) // 16
    one_col = d_out

    w_pad = jnp.zeros((k_in, _DSC), F32).at[:w.shape[0], :d_out].set(w)
    s_pad = jnp.zeros((_DSC, 1), F32).at[:d_out, 0].set(att_s)
    dd_pad = jnp.zeros((_DSC, 1), F32).at[:d_out, 0].set(att_d)
    b_pad = jnp.zeros((1, _DSC), F32).at[0, :d_out].set(bias)

    h, a_s, a_d, bm = _tc_feats(feats, w_pad, s_pad, dd_pad, one_col, n_pad)
    cmax = jnp.maximum(jnp.max(bm[:, 0, 0]) + jnp.max(bm[:, 0, 1]), 0.0)
    cvec = jnp.full((16,), 1.0, F32) * cmax

    p = _sc_logits(a_s.reshape(n_pad), a_d.reshape(n_pad), src, dst, cvec,
                   n_pad, e_pad, e_real)

    accp = _sc_scatter(h.reshape(n_pad * 8, 16), srcoff, dst2, p,
                       n_pad, e_pad, n_cc)
    return _tc_norm(accp, b_pad, n_pad, one_col), d_out


def kernel(x, edge_index, W1, att_src1, att_dst1, b1,
           W2, att_src2, att_dst2, b2):
    n, d_in = x.shape
    e = edge_index.shape[1]
    e_real = e + n
    n_pad = _ceil_to(n, _RB)
    e_pad = _ceil_to(e_real, 1024 * _NC * _NS)
    k_in = _ceil_to(d_in, 8)

    x_pad = jnp.zeros((n_pad, k_in), F32).at[:n, :d_in].set(x)
    loop = jnp.arange(n, dtype=I32)
    zpad = jnp.zeros((e_pad - e_real,), I32)
    src = jnp.concatenate([edge_index[0].astype(I32), loop, zpad])
    dst = jnp.concatenate([edge_index[1].astype(I32), loop, zpad])
    dst2 = dst.reshape(e_pad // 128, 128)
    coff = jnp.arange(7, dtype=I32)[:, None]
    srcoff = (src[None, :] * 8 + coff).reshape(7, e_pad // 128, 128)

    h1, _ = _gat_layer(x_pad, W1, att_src1, att_dst1, b1, src, dst, dst2,
                       srcoff, n_pad, e_pad, e_real)
    h2, d2 = _gat_layer(h1, W2, att_src2, att_dst2, b2, src, dst, dst2,
                        srcoff, n_pad, e_pad, e_real)
    return (h2[:n, :d2], edge_index)


# speedup vs baseline: 21.8620x; 1.0141x over previous
import functools

import jax
import jax.numpy as jnp
from jax import lax
from jax.experimental import pallas as pl
from jax.experimental.pallas import tpu as pltpu
from jax.experimental.pallas import tpu_sc as plsc

F32 = jnp.float32
I32 = jnp.int32

_BCAST_DNUMS = lax.GatherDimensionNumbers(
    offset_dims=(), collapsed_slice_dims=(0,), start_index_map=(0,))


def _splat(vec, lane):
    idx = jnp.full((16, 1), lane, I32)
    return lax.gather(vec, idx, _BCAST_DNUMS, (1,),
                      mode=lax.GatherScatterMode.PROMISE_IN_BOUNDS)

_NC = 2
_NS = 16
_RB = 1024


def _ceil_to(v, m):
    return (v + m - 1) // m * m


_DSC = 128


def _tc_feats(x_in, w_pad, att_s, att_d, one_col, n_pad):
    k_in = x_in.shape[1]
    d_pad = w_pad.shape[1]
    grid = n_pad // _RB

    def body(x_ref, w_ref, s_ref, d_ref, h_ref, as_ref, ad_ref, bm_ref):
        h = jnp.dot(x_ref[...], w_ref[...], preferred_element_type=F32)
        col = lax.broadcasted_iota(I32, (_RB, d_pad), 1)
        h = h + (col == one_col).astype(F32)
        h_ref[...] = h
        a_s = jnp.dot(h, s_ref[...], preferred_element_type=F32)
        a_d = jnp.dot(h, d_ref[...], preferred_element_type=F32)
        as_ref[...] = a_s
        ad_ref[...] = a_d
        i0 = lax.broadcasted_iota(I32, (1, 1, 128), 2)
        bm_ref[...] = jnp.where(i0 == 0, jnp.max(a_s),
                                jnp.where(i0 == 1, jnp.max(a_d), -1e30))

    return pl.pallas_call(
        body,
        grid=(grid,),
        in_specs=[pl.BlockSpec((_RB, k_in), lambda i: (i, 0)),
                  pl.BlockSpec((k_in, d_pad), lambda i: (0, 0)),
                  pl.BlockSpec((d_pad, 1), lambda i: (0, 0)),
                  pl.BlockSpec((d_pad, 1), lambda i: (0, 0))],
        out_specs=[pl.BlockSpec((_RB, d_pad), lambda i: (i, 0)),
                   pl.BlockSpec((_RB, 1), lambda i: (i, 0)),
                   pl.BlockSpec((_RB, 1), lambda i: (i, 0)),
                   pl.BlockSpec((1, 1, 128), lambda i: (i, 0, 0))],
        out_shape=[jax.ShapeDtypeStruct((n_pad, d_pad), F32),
                   jax.ShapeDtypeStruct((n_pad, 1), F32),
                   jax.ShapeDtypeStruct((n_pad, 1), F32),
                   jax.ShapeDtypeStruct((grid, 1, 128), F32)],
    )(x_in, w_pad, att_s, att_d)


def _tc_norm(accp, bias_pad, n_pad, one_col):
    grid = n_pad // _RB

    def body(a_ref, b_ref, o_ref):
        acc = a_ref[0] + a_ref[1]
        col = lax.broadcasted_iota(I32, (_RB, _DSC), 1)
        s = jnp.sum(jnp.where(col == one_col, acc, 0.0), axis=1, keepdims=True)
        out = jnp.maximum(acc / (s + 1e-16) + b_ref[...], 0.0)
        o_ref[...] = jnp.where(col <= one_col, out, 0.0)

    return pl.pallas_call(
        body,
        grid=(grid,),
        in_specs=[pl.BlockSpec((2, _RB, _DSC), lambda i: (0, i, 0)),
                  pl.BlockSpec((1, _DSC), lambda i: (0, 0))],
        out_specs=pl.BlockSpec((_RB, _DSC), lambda i: (i, 0)),
        out_shape=jax.ShapeDtypeStruct((n_pad, _DSC), F32),
    )(accp, bias_pad)


def _tc_norm_feats(accp, bias1, w2_pad, att_s2, att_d2, one1, one2, n_pad):
    grid = n_pad // _RB

    def body(a_ref, b_ref, w_ref, s_ref, d_ref, h_ref, as_ref, ad_ref,
             bm_ref):
        acc = a_ref[0] + a_ref[1]
        col = lax.broadcasted_iota(I32, (_RB, _DSC), 1)
        s = jnp.sum(jnp.where(col == one1, acc, 0.0), axis=1, keepdims=True)
        h1 = jnp.maximum(acc / (s + 1e-16) + b_ref[...], 0.0)
        h1 = jnp.where(col <= one1, h1, 0.0)
        h = jnp.dot(h1, w_ref[...], preferred_element_type=F32)
        h = h + (col == one2).astype(F32)
        h_ref[...] = h
        a_s = jnp.dot(h, s_ref[...], preferred_element_type=F32)
        a_d = jnp.dot(h, d_ref[...], preferred_element_type=F32)
        as_ref[...] = a_s
        ad_ref[...] = a_d
        i0 = lax.broadcasted_iota(I32, (1, 1, 128), 2)
        bm_ref[...] = jnp.where(i0 == 0, jnp.max(a_s),
                                jnp.where(i0 == 1, jnp.max(a_d), -1e30))

    return pl.pallas_call(
        body,
        grid=(grid,),
        in_specs=[pl.BlockSpec((2, _RB, _DSC), lambda i: (0, i, 0)),
                  pl.BlockSpec((1, _DSC), lambda i: (0, 0)),
                  pl.BlockSpec((_DSC, _DSC), lambda i: (0, 0)),
                  pl.BlockSpec((_DSC, 1), lambda i: (0, 0)),
                  pl.BlockSpec((_DSC, 1), lambda i: (0, 0))],
        out_specs=[pl.BlockSpec((_RB, _DSC), lambda i: (i, 0)),
                   pl.BlockSpec((_RB, 1), lambda i: (i, 0)),
                   pl.BlockSpec((_RB, 1), lambda i: (i, 0)),
                   pl.BlockSpec((1, 1, 128), lambda i: (i, 0, 0))],
        out_shape=[jax.ShapeDtypeStruct((n_pad, _DSC), F32),
                   jax.ShapeDtypeStruct((n_pad, 1), F32),
                   jax.ShapeDtypeStruct((n_pad, 1), F32),
                   jax.ShapeDtypeStruct((grid, 1, 128), F32)],
    )(accp, bias1, w2_pad, att_s2, att_d2)


def _sc_logits(a_src, a_dst, src, dst, cvec, n_pad, e_pad, e_real):
    t_per = e_pad // (_NC * _NS)
    n_chunks = t_per // 1024
    mesh = plsc.VectorSubcoreMesh(core_axis_name="c", subcore_axis_name="s",
                                  num_cores=_NC, num_subcores=_NS)

    @functools.partial(
        pl.kernel,
        out_type=jax.ShapeDtypeStruct((e_pad,), F32),
        mesh=mesh,
        compiler_params=pltpu.CompilerParams(needs_layout_passes=False),
        scratch_types=[pltpu.VMEM((n_pad,), F32),
                       pltpu.VMEM((n_pad,), F32),
                       pltpu.VMEM((1024,), I32),
                       pltpu.VMEM((1024,), I32),
                       pltpu.VMEM((1024,), F32),
                       pltpu.VMEM((16,), F32)])
    def k(as_hbm, ad_hbm, src_hbm, dst_hbm, cv_hbm, p_hbm,
          as_v, ad_v, src_v, dst_v, p_v, cv_v):
        cid = lax.axis_index("c")
        sid = lax.axis_index("s")
        wid = cid * _NS + sid
        pltpu.sync_copy(as_hbm, as_v)
        pltpu.sync_copy(ad_hbm, ad_v)
        pltpu.sync_copy(cv_hbm, cv_v)
        cvr = cv_v[...]
        tbase = wid * t_per

        def chunk(g, carry):
            base = tbase + g * 1024
            pltpu.sync_copy(src_hbm.at[pl.ds(base, 1024)], src_v)
            pltpu.sync_copy(dst_hbm.at[pl.ds(base, 1024)], dst_v)

            def grp(i, c2):
                sv = src_v[pl.ds(i * 16, 16)]
                dv = dst_v[pl.ds(i * 16, 16)]
                e = plsc.load_gather(as_v, [sv]) + plsc.load_gather(ad_v, [dv])
                e = jnp.where(e > 0, e, 0.2 * e)
                gi = base + i * 16 + lax.iota(I32, 16)
                p_v[pl.ds(i * 16, 16)] = jnp.where(
                    gi < e_real, jnp.exp(e - cvr), 0.0)
                return c2

            lax.fori_loop(0, 64, grp, 0)
            pltpu.sync_copy(p_v, p_hbm.at[pl.ds(base, 1024)])
            return carry

        lax.fori_loop(0, n_chunks, chunk, 0)

    return k(a_src, a_dst, src, dst, cvec)


def _sc_scatter(hflat, srcoff, dst2, p, n_pad, e_pad, n_cc):
    t_per = e_pad // (_NC * _NS)
    n_chunks = t_per // 1024
    rpt = n_pad // _NS
    zr = 112
    nz = rpt // zr
    mesh = plsc.VectorSubcoreMesh(core_axis_name="c", subcore_axis_name="s",
                                  num_cores=_NC, num_subcores=_NS)

    @functools.partial(
        pl.kernel,
        out_type=jax.ShapeDtypeStruct((_NC, n_pad, _DSC), F32),
        mesh=mesh,
        compiler_params=pltpu.CompilerParams(needs_layout_passes=False,
                                             use_tc_tiling_on_sc=False),
        scratch_types=[pltpu.VMEM((8, 128), I32),
                       pltpu.VMEM((t_per // 128, 128), I32),
                       pltpu.VMEM((t_per,), F32),
                       pltpu.VMEM((128, 16), F32),
                       pltpu.VMEM((128, 16), F32),
                       pltpu.VMEM((128, 16), F32),
                       pltpu.VMEM((zr, 16), F32),
                       pltpu.VMEM_SHARED((n_pad, 16), F32),
                       pltpu.SemaphoreType.DMA,
                       pltpu.SemaphoreType.DMA,
                       pltpu.SemaphoreType.DMA,
                       pltpu.SemaphoreType.DMA,
                       pltpu.SemaphoreType.DMA,
                       pltpu.SemaphoreType.DMA])
    def k(h_hbm, so_hbm, d2_hbm, p_hbm, out_hbm,
          srcb, dstb, pb, rows0, rows1, rows2, zbuf, acc,
          gs0, gs1, gs2, ss0, ss1, ss2):
        rows = [rows0, rows1, rows2]
        gsem = [gs0, gs1, gs2]
        ssem = [ss0, ss1, ss2]
        cid = lax.axis_index("c")
        sid = lax.axis_index("s")
        wid = cid * _NS + sid
        row0 = sid * rpt
        irows = t_per // 128

        def zrow(r, c_):
            zbuf[r] = jnp.zeros((16,), F32)
            return c_

        lax.fori_loop(0, zr, zrow, 0)
        pltpu.sync_copy(d2_hbm.at[pl.ds(wid * irows, irows)], dstb)
        pltpu.sync_copy(p_hbm.at[pl.ds(wid * t_per, t_per)], pb)

        for c in range(n_cc):
            def zcp(z, c_):
                pltpu.sync_copy(zbuf, acc.at[pl.ds(row0 + z * zr, zr)])
                return c_

            lax.fori_loop(0, nz, zcp, 0)
            plsc.subcore_barrier()

            def chunk(g, c_):
                pltpu.sync_copy(
                    so_hbm.at[c, pl.ds(wid * (t_per // 128) + g * 8, 8)],
                    srcb)
                sdesc = [None, None, None]
                gdesc = [None, None, None]
                gdesc[0] = pltpu.async_copy(
                    h_hbm.at[srcb.at[0]], rows[0], gsem[0])
                for j in range(8):
                    b = j % 3
                    gdesc[b].wait()
                    if j + 1 < 8:
                        nb = (j + 1) % 3
                        if sdesc[nb] is not None:
                            sdesc[nb].wait()
                        gdesc[nb] = pltpu.async_copy(
                            h_hbm.at[srcb.at[j + 1]], rows[nb], gsem[nb])

                    def scale(q, c3, j=j, b=b):
                        pv = pb[pl.ds(g * 1024 + j * 128 + q * 16, 16)]
                        for l in range(16):
                            r = q * 16 + l
                            rows[b][r] = rows[b][r] * _splat(pv, l)
                        return c3

                    lax.fori_loop(0, 8, scale, 0)
                    sdesc[b] = pltpu.async_copy(
                        rows[b], acc.at[dstb.at[g * 8 + j]], ssem[b],
                        add=True)
                for b in range(3):
                    if sdesc[b] is not None:
                        sdesc[b].wait()
                return c_

            lax.fori_loop(0, n_chunks, chunk, 0)
            plsc.subcore_barrier()
            pltpu.sync_copy(
                acc.at[pl.ds(row0, rpt)],
                out_hbm.at[cid, pl.ds(row0, rpt), pl.ds(c * 16, 16)])

    return k(hflat, srcoff, dst2, p)


def _pad_w(w, d_out):
    wp = jnp.zeros((_DSC, _DSC), F32) if w.shape[0] > 8 else \
        jnp.zeros((_ceil_to(w.shape[0], 8), _DSC), F32)
    return wp.at[:w.shape[0], :d_out].set(w)


def _pad_att(att, d_out):
    return jnp.zeros((_DSC, 1), F32).at[:d_out, 0].set(att)


def _pad_b(b, d_out):
    return jnp.zeros((1, _DSC), F32).at[0, :d_out].set(b)


def _edge_pass(h, a_s, a_d, bm, src, dst, dst2, srcoff, n_pad, e_pad,
               e_real, d_out):
    n_cc = _ceil_to(d_out + 1, 16) // 16
    cmax = jnp.maximum(jnp.max(bm[:, 0, 0]) + jnp.max(bm[:, 0, 1]), 0.0)
    cvec = jnp.full((16,), 1.0, F32) * cmax
    p = _sc_logits(a_s.reshape(n_pad), a_d.reshape(n_pad), src, dst, cvec,
                   n_pad, e_pad, e_real)
    return _sc_scatter(h.reshape(n_pad * 8, 16), srcoff, dst2, p,
                       n_pad, e_pad, n_cc)


def kernel(x, edge_index, W1, att_src1, att_dst1, b1,
           W2, att_src2, att_dst2, b2):
    n, d_in = x.shape
    e = edge_index.shape[1]
    e_real = e + n
    n_pad = _ceil_to(n, _RB)
    e_pad = _ceil_to(e_real, 1024 * _NC * _NS)
    k_in = _ceil_to(d_in, 8)

    x_pad = jnp.zeros((n_pad, k_in), F32).at[:n, :d_in].set(x)
    loop = jnp.arange(n, dtype=I32)
    zpad = jnp.zeros((e_pad - e_real,), I32)
    src = jnp.concatenate([edge_index[0].astype(I32), loop, zpad])
    dst = jnp.concatenate([edge_index[1].astype(I32), loop, zpad])
    dst2 = dst.reshape(e_pad // 128, 128)
    coff = jnp.arange(7, dtype=I32)[:, None]
    srcoff = (src[None, :] * 8 + coff).reshape(7, e_pad // 128, 128)

    d1 = W1.shape[1]
    d2 = W2.shape[1]
    h1f, as1, ad1, bm1 = _tc_feats(
        x_pad, _pad_w(W1, d1), _pad_att(att_src1, d1), _pad_att(att_dst1, d1),
        d1, n_pad)
    accp1 = _edge_pass(h1f, as1, ad1, bm1, src, dst, dst2, srcoff,
                       n_pad, e_pad, e_real, d1)
    h2f, as2, ad2, bm2 = _tc_norm_feats(
        accp1, _pad_b(b1, d1), _pad_w(W2, d2), _pad_att(att_src2, d2),
        _pad_att(att_dst2, d2), d1, d2, n_pad)
    accp2 = _edge_pass(h2f, as2, ad2, bm2, src, dst, dst2, srcoff,
                       n_pad, e_pad, e_real, d2)
    out = _tc_norm(accp2, _pad_b(b2, d2), n_pad, d2)
    return (out[:n, :d2], edge_index)
